# Initial kernel scaffold; baseline (speedup 1.0000x reference)
import jax
import jax.numpy as jnp
from jax.experimental import pallas as pl


def kernel(weight, bias, weight_parameters, bias_parameters, W_w, b_w, W_b, b_b, W_enc, b_enc):
    raise NotImplementedError("write your pallas kernel here")


# speedup vs baseline: 7.7152x; 7.7152x over previous
import jax
import jax.numpy as jnp
from jax.experimental import pallas as pl


def _fused_kernel(weight_ref, bias_ref, wlen_ref, blen_ref,
                  W_w_ref, b_w_ref, W_b_ref, b_b_ref,
                  W_enc_ref, b_enc_ref, out_ref):
    B, LW = weight_ref.shape
    _, LB = bias_ref.shape

    wlen = wlen_ref[...]
    blen = blen_ref[...]

    mask_w = jax.lax.broadcasted_iota(jnp.int32, (B, LW), 1) < wlen
    s_w = jnp.sum(jnp.where(mask_w, weight_ref[...], 0.0), axis=1, keepdims=True)
    mask_b = jax.lax.broadcasted_iota(jnp.int32, (B, LB), 1) < blen
    s_b = jnp.sum(jnp.where(mask_b, bias_ref[...], 0.0), axis=1, keepdims=True)

    lwf = wlen.astype(jnp.float32)
    lbf = blen.astype(jnp.float32)

    emb_w = jax.nn.sigmoid(s_w * W_w_ref[...][None, :] + lwf * b_w_ref[...][None, :])
    emb_b = jax.nn.sigmoid(s_b * W_b_ref[...][None, :] + lbf * b_b_ref[...][None, :])

    emb = jnp.concatenate([emb_w, emb_b], axis=1)
    enc = jnp.dot(emb, W_enc_ref[...], preferred_element_type=jnp.float32)
    out_ref[...] = jnp.maximum(enc + b_enc_ref[...][None, :], 0.0)


def kernel(weight, bias, weight_parameters, bias_parameters, W_w, b_w, W_b, b_b, W_enc, b_enc):
    B = weight.shape[0]
    H = W_enc.shape[1]
    wlen = weight_parameters.astype(jnp.int32).reshape(B, 1)
    blen = bias_parameters.astype(jnp.int32).reshape(B, 1)
    return pl.pallas_call(
        _fused_kernel,
        out_shape=jax.ShapeDtypeStruct((B, H), jnp.float32),
    )(weight, bias, wlen, blen, W_w, b_w, W_b, b_b, W_enc, b_enc)
